# Initial kernel scaffold; baseline (speedup 1.0000x reference)
import functools

import jax
import jax.numpy as jnp
from jax import lax
from jax.experimental import pallas as pl
from jax.experimental.pallas import tpu as pltpu
from jax.experimental.pallas import tpu_sc as plsc

CHUNK = 128


def _sc_gather(idx2, table):
    n_chunks, ch = idx2.shape
    d = table.shape[1]
    n_rows = n_chunks * ch
    info = plsc.get_sparse_core_info()
    nc, ns = info.num_cores, info.num_subcores
    nw = nc * ns
    chunks_per_w = n_chunks // nw
    mesh = plsc.VectorSubcoreMesh(core_axis_name="c", subcore_axis_name="s")

    @functools.partial(
        pl.kernel,
        mesh=mesh,
        out_type=jax.ShapeDtypeStruct((n_rows, d), jnp.float32),
        scratch_types=[
            pltpu.VMEM((chunks_per_w, ch), jnp.int32),
            pltpu.VMEM((ch, d), jnp.float32),
            pltpu.SemaphoreType.DMA,
        ],
    )
    def gk(idx_hbm, table_hbm, out_hbm, idx_v, rows_v, sem):
        wid = lax.axis_index("s") * nc + lax.axis_index("c")
        pltpu.sync_copy(idx_hbm.at[pl.ds(wid * chunks_per_w, chunks_per_w)], idx_v)

        def body(j, carry):
            pltpu.async_copy(table_hbm.at[idx_v.at[j]], rows_v, sem).wait()
            base = (wid * chunks_per_w + j) * ch
            pltpu.sync_copy(rows_v, out_hbm.at[pl.ds(base, ch)])
            return carry

        lax.fori_loop(0, chunks_per_w, body, 0)

    return gk(idx2, table)


def _tc_linear(emb, W, b):
    n_rows, d = emb.shape
    o = W.shape[0]
    blk = 8192
    grid = (n_rows // blk,)

    def mk(e_ref, w_ref, b_ref, o_ref):
        o_ref[...] = lax.dot_general(
            e_ref[...], w_ref[...], (((1,), (1,)), ((), ())),
            preferred_element_type=jnp.float32) + b_ref[...]

    return pl.pallas_call(
        mk,
        grid=grid,
        in_specs=[
            pl.BlockSpec((blk, d), lambda i: (i, 0)),
            pl.BlockSpec((o, d), lambda i: (0, 0)),
            pl.BlockSpec((1, o), lambda i: (0, 0)),
        ],
        out_specs=pl.BlockSpec((blk, o), lambda i: (i, 0)),
        out_shape=jax.ShapeDtypeStruct((n_rows, o), jnp.float32),
    )(emb, W, b.reshape(1, o))


def kernel(indices, table, W, b):
    batch, hist = indices.shape
    n_rows = batch * hist
    idx2 = indices.reshape(n_rows // CHUNK, CHUNK)
    emb = _sc_gather(idx2, table)
    out = _tc_linear(emb, W, b)
    return out.reshape(batch, hist, W.shape[0])


# speedup vs baseline: 1.0576x; 1.0576x over previous
import functools

import jax
import jax.numpy as jnp
from jax import lax
from jax.experimental import pallas as pl
from jax.experimental.pallas import tpu as pltpu
from jax.experimental.pallas import tpu_sc as plsc

CHUNK = 128


def _sc_gather(idx2, table):
    n_chunks, ch = idx2.shape
    d = table.shape[1]
    n_rows = n_chunks * ch
    info = plsc.get_sparse_core_info()
    nc, ns = info.num_cores, info.num_subcores
    nw = nc * ns
    chunks_per_w = n_chunks // nw
    mesh = plsc.VectorSubcoreMesh(core_axis_name="c", subcore_axis_name="s")

    @functools.partial(
        pl.kernel,
        mesh=mesh,
        compiler_params=pltpu.CompilerParams(use_tc_tiling_on_sc=False),
        out_type=jax.ShapeDtypeStruct((n_rows, d), jnp.float32),
        scratch_types=[
            pltpu.VMEM((chunks_per_w, ch), jnp.int32),
            pltpu.VMEM((ch, d), jnp.float32),
            pltpu.SemaphoreType.DMA,
        ],
    )
    def gk(idx_hbm, table_hbm, out_hbm, idx_v, rows_v, sem):
        wid = lax.axis_index("s") * nc + lax.axis_index("c")
        pltpu.sync_copy(idx_hbm.at[pl.ds(wid * chunks_per_w, chunks_per_w)], idx_v)

        def body(j, carry):
            pltpu.async_copy(table_hbm.at[idx_v.at[j]], rows_v, sem).wait()
            base = (wid * chunks_per_w + j) * ch
            pltpu.sync_copy(rows_v, out_hbm.at[pl.ds(base, ch)])
            return carry

        lax.fori_loop(0, chunks_per_w, body, 0)

    return gk(idx2, table)


def _tc_linear(emb, W, b):
    n_rows, d = emb.shape
    o = W.shape[0]
    blk = 8192
    grid = (n_rows // blk,)

    def mk(e_ref, w_ref, b_ref, o_ref):
        o_ref[...] = lax.dot_general(
            e_ref[...], w_ref[...], (((1,), (1,)), ((), ())),
            preferred_element_type=jnp.float32) + b_ref[...]

    return pl.pallas_call(
        mk,
        grid=grid,
        in_specs=[
            pl.BlockSpec((blk, d), lambda i: (i, 0)),
            pl.BlockSpec((o, d), lambda i: (0, 0)),
            pl.BlockSpec((1, o), lambda i: (0, 0)),
        ],
        out_specs=pl.BlockSpec((blk, o), lambda i: (i, 0)),
        out_shape=jax.ShapeDtypeStruct((n_rows, o), jnp.float32),
    )(emb, W, b.reshape(1, o))


def kernel(indices, table, W, b):
    batch, hist = indices.shape
    n_rows = batch * hist
    idx2 = indices.reshape(n_rows // CHUNK, CHUNK)
    emb = _sc_gather(idx2, table)
    out = _tc_linear(emb, W, b)
    return out.reshape(batch, hist, W.shape[0])


# speedup vs baseline: 4.1946x; 3.9662x over previous
import functools

import jax
import jax.numpy as jnp
from jax import lax
from jax.experimental import pallas as pl
from jax.experimental.pallas import tpu as pltpu
from jax.experimental.pallas import tpu_sc as plsc

CHUNK = 128
K = 2
SUP = CHUNK * K


def _tc_table_linear(tableT, W, b):
    d, v = tableT.shape
    o = W.shape[0]
    blkv = 16384
    grid = (pl.cdiv(v, blkv),)

    def mk(t_ref, w_ref, b_ref, o_ref):
        o_ref[...] = lax.dot_general(
            t_ref[...], w_ref[...], (((0,), (1,)), ((), ())),
            preferred_element_type=jnp.float32) + b_ref[...]

    return pl.pallas_call(
        mk,
        grid=grid,
        in_specs=[
            pl.BlockSpec((d, blkv), lambda i: (0, i)),
            pl.BlockSpec((o, d), lambda i: (0, 0)),
            pl.BlockSpec((1, o), lambda i: (0, 0)),
        ],
        out_specs=pl.BlockSpec((blkv, o), lambda i: (i, 0)),
        out_shape=jax.ShapeDtypeStruct((v, o), jnp.float32),
    )(tableT, W, b.reshape(1, o))


def _sc_gather(idx2, t2):
    n_chunks, ch = idx2.shape
    d = t2.shape[1]
    n_rows = n_chunks * ch
    info = plsc.get_sparse_core_info()
    nc, ns = info.num_cores, info.num_subcores
    nw = nc * ns
    chunks_per_w = n_chunks // nw
    sup_per_w = chunks_per_w // K
    mesh = plsc.VectorSubcoreMesh(core_axis_name="c", subcore_axis_name="s")

    @functools.partial(
        pl.kernel,
        mesh=mesh,
        out_type=jax.ShapeDtypeStruct((n_rows, d), jnp.float32),
        scratch_types=[
            pltpu.VMEM((chunks_per_w, ch), jnp.int32),
            pltpu.VMEM((SUP, d), jnp.float32),
            pltpu.VMEM((SUP, d), jnp.float32),
            pltpu.SemaphoreType.DMA,
            pltpu.SemaphoreType.DMA,
            pltpu.SemaphoreType.DMA,
            pltpu.SemaphoreType.DMA,
        ],
    )
    def gk(idx_hbm, t2_hbm, out_hbm, idx_v, rows0, rows1, g0, g1, w0, w1):
        wid = lax.axis_index("s") * nc + lax.axis_index("c")
        pltpu.sync_copy(idx_hbm.at[pl.ds(wid * chunks_per_w, chunks_per_w)], idx_v)
        rows = (rows0, rows1)
        gsem = (g0, g1)
        wsem = (w0, w1)
        out_base = wid * chunks_per_w * ch

        def fire(i, x):
            for j in range(K):
                pltpu.make_async_copy(
                    t2_hbm.at[idx_v.at[i * K + j]],
                    rows[x].at[pl.ds(j * CHUNK, CHUNK)],
                    gsem[x]).start()

        def drain(i, x):
            for j in range(K):
                pltpu.make_async_copy(
                    t2_hbm.at[idx_v.at[i * K + j]],
                    rows[x].at[pl.ds(j * CHUNK, CHUNK)],
                    gsem[x]).wait()

        def write(i, x):
            pltpu.make_async_copy(
                rows[x], out_hbm.at[pl.ds(out_base + i * SUP, SUP)], wsem[x]).start()

        def wait_write(i, x):
            pltpu.make_async_copy(
                rows[x], out_hbm.at[pl.ds(out_base + i * SUP, SUP)], wsem[x]).wait()

        fire(0, 0)
        fire(1, 1)
        drain(0, 0)
        write(0, 0)

        def body(p, carry):
            i0 = 2 * p
            wait_write(i0 - 2, 0)
            fire(i0, 0)
            drain(i0 - 1, 1)
            write(i0 - 1, 1)
            i1 = 2 * p + 1
            wait_write(i1 - 2, 1)
            fire(i1, 1)
            drain(i1 - 1, 0)
            write(i1 - 1, 0)
            return carry

        lax.fori_loop(1, sup_per_w // 2, body, 0)
        last = sup_per_w - 1
        drain(last, 1)
        write(last, 1)
        wait_write(last - 1, 0)
        wait_write(last, 1)

    return gk(idx2, t2)


def kernel(indices, table, W, b):
    batch, hist = indices.shape
    n_rows = batch * hist
    t2 = _tc_table_linear(table.T, W, b)
    idx2 = indices.T.reshape(n_rows // CHUNK, CHUNK)
    out = _sc_gather(idx2, t2)
    return out.reshape(hist, batch, W.shape[0]).transpose(1, 0, 2)


# speedup vs baseline: 4.2017x; 1.0017x over previous
import functools

import jax
import jax.numpy as jnp
from jax import lax
from jax.experimental import pallas as pl
from jax.experimental.pallas import tpu as pltpu
from jax.experimental.pallas import tpu_sc as plsc

CHUNK = 128
NBUF = 5


def _tc_table_linear(tableT, W, b):
    d, v = tableT.shape
    o = W.shape[0]
    blkv = 16384
    grid = (pl.cdiv(v, blkv),)

    def mk(t_ref, w_ref, b_ref, o_ref):
        o_ref[...] = lax.dot_general(
            t_ref[...], w_ref[...], (((0,), (1,)), ((), ())),
            preferred_element_type=jnp.float32) + b_ref[...]

    return pl.pallas_call(
        mk,
        grid=grid,
        in_specs=[
            pl.BlockSpec((d, blkv), lambda i: (0, i)),
            pl.BlockSpec((o, d), lambda i: (0, 0)),
            pl.BlockSpec((1, o), lambda i: (0, 0)),
        ],
        out_specs=pl.BlockSpec((blkv, o), lambda i: (i, 0)),
        out_shape=jax.ShapeDtypeStruct((v, o), jnp.float32),
    )(tableT, W, b.reshape(1, o))


def _sc_gather(idx2, t2):
    n_chunks, ch = idx2.shape
    d = t2.shape[1]
    n_rows = n_chunks * ch
    info = plsc.get_sparse_core_info()
    nc, ns = info.num_cores, info.num_subcores
    nw = nc * ns
    chunks_per_w = n_chunks // nw
    mesh = plsc.VectorSubcoreMesh(core_axis_name="c", subcore_axis_name="s")
    assert chunks_per_w % NBUF == 0 and chunks_per_w // NBUF >= 3

    @functools.partial(
        pl.kernel,
        mesh=mesh,
        out_type=jax.ShapeDtypeStruct((n_rows, d), jnp.float32),
        scratch_types=[
            pltpu.VMEM((chunks_per_w, ch), jnp.int32),
            [pltpu.VMEM((ch, d), jnp.float32)] * NBUF,
            [pltpu.SemaphoreType.DMA] * NBUF,
            [pltpu.SemaphoreType.DMA] * NBUF,
        ],
    )
    def gk(idx_hbm, t2_hbm, out_hbm, idx_v, rows, gsem, wsem):
        wid = lax.axis_index("s") * nc + lax.axis_index("c")
        pltpu.sync_copy(idx_hbm.at[pl.ds(wid * chunks_per_w, chunks_per_w)], idx_v)
        out_base = wid * chunks_per_w * ch
        n = chunks_per_w

        def fire(i, x):
            pltpu.make_async_copy(t2_hbm.at[idx_v.at[i]], rows[x], gsem[x]).start()

        def drain(i, x):
            pltpu.make_async_copy(t2_hbm.at[idx_v.at[i]], rows[x], gsem[x]).wait()

        def write(i, x):
            pltpu.make_async_copy(
                rows[x], out_hbm.at[pl.ds(out_base + i * ch, ch)], wsem[x]).start()

        def wait_write(i, x):
            pltpu.make_async_copy(
                rows[x], out_hbm.at[pl.ds(out_base + i * ch, ch)], wsem[x]).wait()

        for j in range(NBUF - 1):
            fire(j, j)

        def step(i, x):
            drain(i, x)
            write(i, x)
            y = (x - 1) % NBUF
            wait_write(i - 1, y)
            fire(i + NBUF - 1, y)

        drain(0, 0)
        write(0, 0)
        fire(NBUF - 1, NBUF - 1)
        for u in range(1, NBUF):
            step(u, u)

        def body(q, carry):
            for u in range(NBUF):
                step(q * NBUF + u, u)
            return carry

        lax.fori_loop(1, n // NBUF - 1, body, 0)
        tail = n - NBUF
        step(tail, 0)
        for u in range(1, NBUF):
            i = tail + u
            drain(i, u)
            write(i, u)
            wait_write(i - 1, (u - 1) % NBUF)
        wait_write(n - 1, (NBUF - 1) % NBUF)

    return gk(idx2, t2)


def kernel(indices, table, W, b):
    batch, hist = indices.shape
    n_rows = batch * hist
    t2 = _tc_table_linear(table.T, W, b)
    idx2 = indices.T.reshape(n_rows // CHUNK, CHUNK)
    out = _sc_gather(idx2, t2)
    return out.reshape(hist, batch, W.shape[0]).transpose(1, 0, 2)


# speedup vs baseline: 4.2070x; 1.0013x over previous
import functools

import jax
import jax.numpy as jnp
from jax import lax
from jax.experimental import pallas as pl
from jax.experimental.pallas import tpu as pltpu
from jax.experimental.pallas import tpu_sc as plsc

CHUNK = 128
NBUF = 5


def _tc_table_linear(tableT, W, b):
    d, v = tableT.shape
    o = W.shape[0]
    blkv = 16384
    grid = (pl.cdiv(v, blkv),)

    def mk(t_ref, w_ref, b_ref, o_ref):
        o_ref[...] = lax.dot_general(
            t_ref[...], w_ref[...], (((0,), (1,)), ((), ())),
            preferred_element_type=jnp.float32) + b_ref[...]

    return pl.pallas_call(
        mk,
        grid=grid,
        in_specs=[
            pl.BlockSpec((d, blkv), lambda i: (0, i)),
            pl.BlockSpec((o, d), lambda i: (0, 0)),
            pl.BlockSpec((1, o), lambda i: (0, 0)),
        ],
        out_specs=pl.BlockSpec((blkv, o), lambda i: (i, 0)),
        out_shape=jax.ShapeDtypeStruct((v, o), jnp.float32),
    )(tableT, W, b.reshape(1, o))


def _sc_gather(idx2, t2):
    n_chunks, ch = idx2.shape
    d = t2.shape[1]
    n_rows = n_chunks * ch
    info = plsc.get_sparse_core_info()
    nc, ns = info.num_cores, info.num_subcores
    nw = nc * ns
    chunks_per_w = n_chunks // nw
    mesh = plsc.VectorSubcoreMesh(core_axis_name="c", subcore_axis_name="s")
    assert chunks_per_w % NBUF == 0 and chunks_per_w // NBUF >= 3

    @functools.partial(
        pl.kernel,
        mesh=mesh,
        out_type=jax.ShapeDtypeStruct((n_rows, d), jnp.float32),
        scratch_types=[
            pltpu.VMEM((chunks_per_w, ch), jnp.int32),
            [pltpu.VMEM((ch, d), jnp.float32)] * NBUF,
            [pltpu.SemaphoreType.DMA] * NBUF,
            [pltpu.SemaphoreType.DMA] * NBUF,
        ],
    )
    def gk(idx_hbm, t2_hbm, out_hbm, idx_v, rows, gsem, wsem):
        wid = lax.axis_index("s") * nc + lax.axis_index("c")
        pltpu.sync_copy(idx_hbm.at[pl.ds(wid * chunks_per_w, chunks_per_w)], idx_v)
        out_base = wid * chunks_per_w * ch
        n = chunks_per_w

        def fire(i, x):
            pltpu.make_async_copy(t2_hbm.at[idx_v.at[i]], rows[x], gsem[x]).start()

        def drain(i, x):
            pltpu.make_async_copy(t2_hbm.at[idx_v.at[i]], rows[x], gsem[x]).wait()

        def write(i, x):
            pltpu.make_async_copy(
                rows[x], out_hbm.at[pl.ds(out_base + i * ch, ch)], wsem[x]).start()

        def wait_write(i, x):
            pltpu.make_async_copy(
                rows[x], out_hbm.at[pl.ds(out_base + i * ch, ch)], wsem[x]).wait()

        G = 3
        S = NBUF - G

        def step(i, x):
            drain(i, x)
            write(i, x)
            y = (x + G) % NBUF
            wait_write(i - S, y)
            fire(i + G, y)

        for j in range(G):
            fire(j, j)
        for u in range(NBUF):
            drain(u, u)
            write(u, u)
            if u >= S:
                wait_write(u - S, (u + G) % NBUF)
            fire(u + G, (u + G) % NBUF)

        def body(q, carry):
            for u in range(NBUF):
                step(q * NBUF + u, u)
            return carry

        lax.fori_loop(1, n // NBUF - 1, body, 0)
        tail = n - NBUF
        for u in range(NBUF):
            i = tail + u
            drain(i, u)
            write(i, u)
            wait_write(i - S, (u + G) % NBUF)
            if u < NBUF - G:
                fire(i + G, (u + G) % NBUF)
        for j in range(S):
            wait_write(n - S + j, (n - S + j) % NBUF)

    return gk(idx2, t2)


def kernel(indices, table, W, b):
    batch, hist = indices.shape
    n_rows = batch * hist
    t2 = _tc_table_linear(table.T, W, b)
    idx2 = indices.T.reshape(n_rows // CHUNK, CHUNK)
    out = _sc_gather(idx2, t2)
    return out.reshape(hist, batch, W.shape[0]).transpose(1, 0, 2)


# speedup vs baseline: 4.2493x; 1.0101x over previous
import functools

import jax
import jax.numpy as jnp
from jax import lax
from jax.experimental import pallas as pl
from jax.experimental.pallas import tpu as pltpu
from jax.experimental.pallas import tpu_sc as plsc

CHUNK = 128
NBUF = 5


def _tc_table_linear(tableT, W, b):
    d, v = tableT.shape
    o = W.shape[0]
    blkv = 32768
    grid = (pl.cdiv(v, blkv),)

    def mk(t_ref, w_ref, b_ref, o_ref):
        o_ref[...] = lax.dot_general(
            t_ref[...], w_ref[...], (((0,), (1,)), ((), ())),
            preferred_element_type=jnp.float32) + b_ref[...]

    return pl.pallas_call(
        mk,
        grid=grid,
        in_specs=[
            pl.BlockSpec((d, blkv), lambda i: (0, i)),
            pl.BlockSpec((o, d), lambda i: (0, 0)),
            pl.BlockSpec((1, o), lambda i: (0, 0)),
        ],
        out_specs=pl.BlockSpec((blkv, o), lambda i: (i, 0)),
        out_shape=jax.ShapeDtypeStruct((v, o), jnp.float32),
    )(tableT, W, b.reshape(1, o))


def _sc_gather(idx2, t2):
    n_chunks, ch = idx2.shape
    d = t2.shape[1]
    n_rows = n_chunks * ch
    info = plsc.get_sparse_core_info()
    nc, ns = info.num_cores, info.num_subcores
    nw = nc * ns
    chunks_per_w = n_chunks // nw
    mesh = plsc.VectorSubcoreMesh(core_axis_name="c", subcore_axis_name="s")
    assert chunks_per_w % NBUF == 0 and chunks_per_w // NBUF >= 3

    @functools.partial(
        pl.kernel,
        mesh=mesh,
        out_type=jax.ShapeDtypeStruct((n_rows, d), jnp.float32),
        scratch_types=[
            pltpu.VMEM((chunks_per_w, ch), jnp.int32),
            [pltpu.VMEM((ch, d), jnp.float32)] * NBUF,
            [pltpu.SemaphoreType.DMA] * NBUF,
            [pltpu.SemaphoreType.DMA] * NBUF,
        ],
    )
    def gk(idx_hbm, t2_hbm, out_hbm, idx_v, rows, gsem, wsem):
        wid = lax.axis_index("s") * nc + lax.axis_index("c")
        pltpu.sync_copy(idx_hbm.at[pl.ds(wid * chunks_per_w, chunks_per_w)], idx_v)
        out_base = wid * chunks_per_w * ch
        n = chunks_per_w

        def fire(i, x):
            pltpu.make_async_copy(t2_hbm.at[idx_v.at[i]], rows[x], gsem[x]).start()

        def drain(i, x):
            pltpu.make_async_copy(t2_hbm.at[idx_v.at[i]], rows[x], gsem[x]).wait()

        def write(i, x):
            pltpu.make_async_copy(
                rows[x], out_hbm.at[pl.ds(out_base + i * ch, ch)], wsem[x]).start()

        def wait_write(i, x):
            pltpu.make_async_copy(
                rows[x], out_hbm.at[pl.ds(out_base + i * ch, ch)], wsem[x]).wait()

        G = 3
        S = NBUF - G

        def step(i, x):
            drain(i, x)
            write(i, x)
            y = (x + G) % NBUF
            wait_write(i - S, y)
            fire(i + G, y)

        for j in range(G):
            fire(j, j)
        for u in range(NBUF):
            drain(u, u)
            write(u, u)
            if u >= S:
                wait_write(u - S, (u + G) % NBUF)
            fire(u + G, (u + G) % NBUF)

        def body(q, carry):
            for u in range(NBUF):
                step(q * NBUF + u, u)
            return carry

        lax.fori_loop(1, n // NBUF - 1, body, 0)
        tail = n - NBUF
        for u in range(NBUF):
            i = tail + u
            drain(i, u)
            write(i, u)
            wait_write(i - S, (u + G) % NBUF)
            if u < NBUF - G:
                fire(i + G, (u + G) % NBUF)
        for j in range(S):
            wait_write(n - S + j, (n - S + j) % NBUF)

    return gk(idx2, t2)


def kernel(indices, table, W, b):
    batch, hist = indices.shape
    n_rows = batch * hist
    t2 = _tc_table_linear(table.T, W, b)
    idx2 = indices.T.reshape(n_rows // CHUNK, CHUNK)
    out = _sc_gather(idx2, t2)
    return out.reshape(hist, batch, W.shape[0]).transpose(1, 0, 2)
